# Initial kernel scaffold; baseline (speedup 1.0000x reference)
import jax
import jax.numpy as jnp
from jax.experimental import pallas as pl


def kernel(input, pool_idx):
    raise NotImplementedError("write your pallas kernel here")


# speedup vs baseline: 8.0174x; 8.0174x over previous
import functools

import jax
import jax.numpy as jnp
from jax import lax
from jax.experimental import pallas as pl
from jax.experimental.pallas import tpu as pltpu
from jax.experimental.pallas import tpu_sc as plsc

N_NODES = 10000
N_EDGES = 160000
D_FEAT = 128

NC = 2
NS = 16
NW = NC * NS

CHUNK = 128
NUM_UNITS = N_EDGES // CHUNK
NUM_ROUNDS = -(-NUM_UNITS // NW)

ROWS_PER_SUBCORE = N_NODES // NS
COPY_ROWS = N_NODES // NW


def _pool_body(x_hbm, ia_hbm, ib_hbm, out_hbm, x_sp, ia_v, ib_v, a_v, b_v, o_v, sem):
    cid = lax.axis_index("c")
    sid = lax.axis_index("s")
    wid = sid * NC + cid

    pltpu.sync_copy(
        x_hbm.at[pl.ds(sid * ROWS_PER_SUBCORE, ROWS_PER_SUBCORE)],
        x_sp.at[pl.ds(sid * ROWS_PER_SUBCORE, ROWS_PER_SUBCORE)],
    )
    plsc.subcore_barrier()

    base = wid * COPY_ROWS + jnp.minimum(wid, 16)
    pltpu.sync_copy(x_sp.at[pl.ds(base, COPY_ROWS)], out_hbm.at[pl.ds(base, COPY_ROWS)])

    @pl.when(wid < 16)
    def _():
        extra = wid * (COPY_ROWS + 1) + COPY_ROWS
        pltpu.sync_copy(x_sp.at[pl.ds(extra, 1)], out_hbm.at[pl.ds(extra, 1)])

    def round_body(r, carry):
        u = r * NW + wid

        @pl.when(u < NUM_UNITS)
        def _():
            off = u * CHUNK
            pltpu.sync_copy(ia_hbm.at[pl.ds(off, CHUNK)], ia_v)
            pltpu.sync_copy(ib_hbm.at[pl.ds(off, CHUNK)], ib_v)
            ca = pltpu.async_copy(x_sp.at[ia_v], a_v, sem)
            cb = pltpu.async_copy(x_sp.at[ib_v], b_v, sem)
            ca.wait()
            cb.wait()

            def avg_body(i, c):
                for v in range(D_FEAT // 16):
                    s = pl.ds(v * 16, 16)
                    o_v[i, s] = (a_v[i, s] + b_v[i, s]) * 0.5
                return c

            lax.fori_loop(0, CHUNK, avg_body, 0)
            pltpu.sync_copy(o_v, out_hbm.at[pl.ds(N_NODES + off, CHUNK)])

        return carry

    lax.fori_loop(0, NUM_ROUNDS, round_body, 0)


@functools.partial(jax.jit, static_argnames=())
def kernel(input, pool_idx):
    idx_t = pool_idx.T.astype(jnp.int32)
    mesh = plsc.VectorSubcoreMesh(
        core_axis_name="c", subcore_axis_name="s", num_cores=NC, num_subcores=NS
    )
    run = pl.kernel(
        _pool_body,
        out_type=jax.ShapeDtypeStruct((N_NODES + N_EDGES, D_FEAT), jnp.float32),
        mesh=mesh,
        compiler_params=pltpu.CompilerParams(use_tc_tiling_on_sc=False),
        scratch_types=[
            pltpu.VMEM_SHARED((N_NODES, D_FEAT), jnp.float32),
            pltpu.VMEM((CHUNK,), jnp.int32),
            pltpu.VMEM((CHUNK,), jnp.int32),
            pltpu.VMEM((CHUNK, D_FEAT), jnp.float32),
            pltpu.VMEM((CHUNK, D_FEAT), jnp.float32),
            pltpu.VMEM((CHUNK, D_FEAT), jnp.float32),
            pltpu.SemaphoreType.DMA,
        ],
    )
    return run(input, idx_t[0], idx_t[1])


# speedup vs baseline: 15.0861x; 1.8817x over previous
import functools

import jax
import jax.numpy as jnp
from jax import lax
from jax.experimental import pallas as pl
from jax.experimental.pallas import tpu as pltpu
from jax.experimental.pallas import tpu_sc as plsc

N_NODES = 10000
N_EDGES = 160000
D_FEAT = 128

NC = 2
NS = 16
NW = NC * NS

CHUNK = 64
NUM_UNITS = N_EDGES // CHUNK
NUM_ROUNDS = 2 * (-(-NUM_UNITS // (2 * NW)))

ROWS_PER_SUBCORE = N_NODES // NS
COPY_ROWS = N_NODES // NW


def _pool_body(x_hbm, ia_hbm, ib_hbm, out_hbm, x_sp, ia_v, ib_v, a_v, b_v, o_v,
               isem0, isem1, gsem0, gsem1, osem0, osem1):
    cid = lax.axis_index("c")
    sid = lax.axis_index("s")
    wid = sid * NC + cid
    idx_sems = (isem0, isem1)
    gat_sems = (gsem0, gsem1)
    out_sems = (osem0, osem1)

    def u_of(r):
        return r * NW + wid

    def act(r):
        return u_of(r) < NUM_UNITS

    def start_idx(r, slot):
        @pl.when(act(r))
        def _():
            off = u_of(r) * CHUNK
            pltpu.async_copy(ia_hbm.at[pl.ds(off, CHUNK)], ia_v.at[slot], idx_sems[slot])
            pltpu.async_copy(ib_hbm.at[pl.ds(off, CHUNK)], ib_v.at[slot], idx_sems[slot])

    def wait_idx(r, slot):
        @pl.when(act(r))
        def _():
            off = u_of(r) * CHUNK
            pltpu.make_async_copy(ia_hbm.at[pl.ds(off, CHUNK)], ia_v.at[slot], idx_sems[slot]).wait()
            pltpu.make_async_copy(ib_hbm.at[pl.ds(off, CHUNK)], ib_v.at[slot], idx_sems[slot]).wait()

    def start_gather(r, slot):
        @pl.when(act(r))
        def _():
            pltpu.async_copy(x_sp.at[ia_v.at[slot]], a_v.at[slot], gat_sems[slot])
            pltpu.async_copy(x_sp.at[ib_v.at[slot]], b_v.at[slot], gat_sems[slot])

    def wait_gather(r, slot):
        @pl.when(act(r))
        def _():
            pltpu.make_async_copy(x_sp.at[ia_v.at[slot]], a_v.at[slot], gat_sems[slot]).wait()
            pltpu.make_async_copy(x_sp.at[ib_v.at[slot]], b_v.at[slot], gat_sems[slot]).wait()

    def start_out(r, slot):
        @pl.when(act(r))
        def _():
            off = u_of(r) * CHUNK
            pltpu.async_copy(o_v.at[slot], out_hbm.at[pl.ds(N_NODES + off, CHUNK)], out_sems[slot])

    def wait_out(r, slot):
        @pl.when((r >= 0) & act(r))
        def _():
            off = u_of(jnp.maximum(r, 0)) * CHUNK
            pltpu.make_async_copy(o_v.at[slot], out_hbm.at[pl.ds(N_NODES + off, CHUNK)], out_sems[slot]).wait()

    def compute(r, slot):
        @pl.when(act(r))
        def _():
            def avg_body(i, c):
                for v in range(D_FEAT // 16):
                    s = pl.ds(v * 16, 16)
                    o_v[slot, i, s] = (a_v[slot, i, s] + b_v[slot, i, s]) * 0.5
                return c

            lax.fori_loop(0, CHUNK, avg_body, 0)

    start_idx(0, 0)
    start_idx(1, 1)
    pltpu.sync_copy(
        x_hbm.at[pl.ds(sid * ROWS_PER_SUBCORE, ROWS_PER_SUBCORE)],
        x_sp.at[pl.ds(sid * ROWS_PER_SUBCORE, ROWS_PER_SUBCORE)],
    )
    plsc.subcore_barrier()

    wait_idx(0, 0)
    start_gather(0, 0)

    def pair_body(r0, carry):
        for slot in (0, 1):
            r = 2 * r0 + slot
            wait_gather(r, slot)
            wait_idx(r + 1, 1 - slot)
            start_gather(r + 1, 1 - slot)
            start_idx(r + 2, slot)
            wait_out(r - 2, slot)
            compute(r, slot)
            start_out(r, slot)
        return carry

    lax.fori_loop(0, NUM_ROUNDS // 2, pair_body, 0)
    wait_out(NUM_ROUNDS - 2, 0)
    wait_out(NUM_ROUNDS - 1, 1)

    base = wid * COPY_ROWS + jnp.minimum(wid, 16)
    pltpu.sync_copy(x_sp.at[pl.ds(base, COPY_ROWS)], out_hbm.at[pl.ds(base, COPY_ROWS)])

    @pl.when(wid < 16)
    def _():
        extra = wid * (COPY_ROWS + 1) + COPY_ROWS
        pltpu.sync_copy(x_sp.at[pl.ds(extra, 1)], out_hbm.at[pl.ds(extra, 1)])


@functools.partial(jax.jit, static_argnames=())
def kernel(input, pool_idx):
    idx_t = pool_idx.T.astype(jnp.int32)
    mesh = plsc.VectorSubcoreMesh(
        core_axis_name="c", subcore_axis_name="s", num_cores=NC, num_subcores=NS
    )
    run = pl.kernel(
        _pool_body,
        out_type=jax.ShapeDtypeStruct((N_NODES + N_EDGES, D_FEAT), jnp.float32),
        mesh=mesh,
        compiler_params=pltpu.CompilerParams(use_tc_tiling_on_sc=False),
        scratch_types=[
            pltpu.VMEM_SHARED((N_NODES, D_FEAT), jnp.float32),
            pltpu.VMEM((2, CHUNK), jnp.int32),
            pltpu.VMEM((2, CHUNK), jnp.int32),
            pltpu.VMEM((2, CHUNK, D_FEAT), jnp.float32),
            pltpu.VMEM((2, CHUNK, D_FEAT), jnp.float32),
            pltpu.VMEM((2, CHUNK, D_FEAT), jnp.float32),
            pltpu.SemaphoreType.DMA,
            pltpu.SemaphoreType.DMA,
            pltpu.SemaphoreType.DMA,
            pltpu.SemaphoreType.DMA,
            pltpu.SemaphoreType.DMA,
            pltpu.SemaphoreType.DMA,
        ],
    )
    return run(input, idx_t[0], idx_t[1])
